# baseline (speedup 1.0000x reference)
import functools

import jax
import jax.numpy as jnp
from jax import lax
from jax.experimental import pallas as pl
from jax.experimental.pallas import tpu as pltpu
from jax.experimental.pallas import tpu_sc as plsc

NNODE = 50000
KORD = 5
NEDGE = 800000

NTILE = 16
NCORE = 2
LANES = 16

NPAD = 50176
DEAD = NNODE
CHUNK = 128
EPAD = 802816
NCHUNK = EPAD // CHUNK
QCOL = 16
NQ = 4
CPT = NCHUNK // NTILE
CPG = 8
GPT = CPT // CPG
RPT = NPAD // NTILE
DR32 = 224
VPC = CHUNK // LANES

_MESH = plsc.VectorSubcoreMesh(
    core_axis_name="c", subcore_axis_name="s",
    num_cores=NCORE, num_subcores=NTILE)
_SC_PARAMS = pltpu.CompilerParams(
    needs_layout_passes=False, use_tc_tiling_on_sc=False)

_f32 = jnp.float32
_i32 = jnp.int32


def _zero_fill(ref, nrows, ncols):
    z = jnp.zeros((LANES,), _f32)
    def row(r, carry):
        for v in range(ncols // LANES):
            ref[r, pl.ds(v * LANES, LANES)] = z
        return carry
    lax.fori_loop(0, nrows, row, 0)


_RCPG = 28
_RGPT = CPT // _RCPG


def _zero_vec(ref, nwords):
    zero = jnp.zeros((LANES,), _f32)
    def zrow(r, carry):
        for u in range(8):
            ref[pl.ds((r * 8 + u) * LANES, LANES)] = zero
        return carry
    lax.fori_loop(0, nwords // (8 * LANES), zrow, 0)


def _reduce_stage(stage, abuf, tbuf, r0):
    pltpu.sync_copy(stage.at[0, pl.ds(r0, RPT)], abuf)
    for t in range(1, NTILE):
        pltpu.sync_copy(stage.at[t, pl.ds(r0, RPT)], tbuf)
        def arow(r, carry):
            sl = pl.ds(r * LANES, LANES)
            abuf[sl] = abuf[sl] + tbuf[sl]
            return carry
        lax.fori_loop(0, RPT // LANES, arow, 0)


@functools.partial(
    pl.kernel,
    out_type=(jax.ShapeDtypeStruct((NPAD,), _f32),
              jax.ShapeDtypeStruct((NTILE, NPAD), _f32)),
    mesh=_MESH,
    name="sc_deg",
    compiler_params=_SC_PARAMS,
    scratch_types=[
        pltpu.VMEM((_RCPG, CHUNK), _i32),
        pltpu.VMEM((_RCPG, CHUNK), _i32),
        pltpu.VMEM((NPAD,), _f32),
        pltpu.VMEM((RPT,), _f32),
        pltpu.VMEM((RPT,), _f32),
    ],
)
def _deg_kernel(src_hbm, dst_hbm, deg_hbm, stage,
                sbuf, dbuf, accl, abuf, tbuf):
    cid = lax.axis_index("c")
    sid = lax.axis_index("s")
    on0 = cid == 0

    @pl.when(on0)
    def _main():
        _zero_vec(accl, NPAD)
        ones = jnp.ones((LANES,), _f32)
        def group(g, carry):
            base = sid * CPT + g * _RCPG
            pltpu.sync_copy(src_hbm.at[pl.ds(base, _RCPG)], sbuf)
            pltpu.sync_copy(dst_hbm.at[pl.ds(base, _RCPG)], dbuf)
            for c in range(_RCPG):
                for v in range(VPC):
                    sl = pl.ds(v * LANES, LANES)
                    s = sbuf[c, sl]
                    d = dbuf[c, sl]
                    plsc.addupdate_scatter(
                        accl, [jnp.where(s != d, s, DEAD)], ones)
            return carry
        lax.fori_loop(0, _RGPT, group, 0)
        pltpu.sync_copy(accl, stage.at[sid])

    plsc.subcore_barrier()

    @pl.when(on0)
    def _drain():
        r0 = sid * RPT
        _reduce_stage(stage, abuf, tbuf, r0)
        pltpu.sync_copy(abuf, deg_hbm.at[pl.ds(r0, RPT)])


_B_NW = 28
_B_CPW = NCHUNK // _B_NW
_B_CPG = 8
_B_GPW = _B_CPW // _B_CPG
_B_NPW = NPAD // (NCORE * NTILE)

@functools.partial(
    pl.kernel,
    out_type=(jax.ShapeDtypeStruct((NQ * NCHUNK, CHUNK), _i32),
              jax.ShapeDtypeStruct((NCHUNK, CHUNK), _i32),
              jax.ShapeDtypeStruct((NPAD,), _f32)),
    mesh=_MESH,
    name="sc_prep",
    compiler_params=_SC_PARAMS,
    scratch_types=[
        pltpu.VMEM((_B_CPG, CHUNK), _i32),
        pltpu.VMEM((_B_CPG, CHUNK), _i32),
        pltpu.VMEM((_B_CPG, CHUNK), _i32),
        pltpu.VMEM((_B_NPW,), _f32),
        pltpu.VMEM((_B_NPW,), _f32),
        pltpu.VMEM((_B_NPW,), _f32),
    ],
)
def _prep_kernel(src_hbm, dst_hbm, dis_hbm, x_hbm,
                 srcef_hbm, dstef_hbm, u0_hbm,
                 sbuf, dbuf, hbuf, xbuf, disbuf, ubuf):
    cid = lax.axis_index("c")
    sid = lax.axis_index("s")
    wid = sid * NCORE + cid

    @pl.when(wid < _B_NW)
    def _edges():
        def group(g, carry):
            base = wid * _B_CPW + g * _B_CPG
            pltpu.sync_copy(src_hbm.at[pl.ds(base, _B_CPG)], sbuf)
            pltpu.sync_copy(dst_hbm.at[pl.ds(base, _B_CPG)], dbuf)
            pltpu.sync_copy(sbuf, srcef_hbm.at[pl.ds(base, _B_CPG)])
            for qq in range(1, NQ):
                for c in range(_B_CPG):
                    for v in range(VPC):
                        sl = pl.ds(v * LANES, LANES)
                        hbuf[c, sl] = sbuf[c, sl] + qq * NPAD
                pltpu.sync_copy(
                    hbuf, srcef_hbm.at[pl.ds(qq * NCHUNK + base, _B_CPG)])
            for c in range(_B_CPG):
                for v in range(VPC):
                    sl = pl.ds(v * LANES, LANES)
                    s = sbuf[c, sl]
                    d = dbuf[c, sl]
                    dbuf[c, sl] = jnp.where(s != d, d, DEAD)
            pltpu.sync_copy(dbuf, dstef_hbm.at[pl.ds(base, _B_CPG)])
            return carry
        lax.fori_loop(0, _B_GPW, group, 0)

    nbase = wid * _B_NPW
    pltpu.sync_copy(x_hbm.at[pl.ds(nbase, _B_NPW)], xbuf)
    pltpu.sync_copy(dis_hbm.at[pl.ds(nbase, _B_NPW)], disbuf)
    def nrow(r, carry):
        sl = pl.ds(r * LANES, LANES)
        ubuf[sl] = xbuf[sl] * disbuf[sl]
        return carry
    lax.fori_loop(0, _B_NPW // LANES, nrow, 0)
    pltpu.sync_copy(ubuf, u0_hbm.at[pl.ds(nbase, _B_NPW)])


def _make_p1(first):
    @functools.partial(
        pl.kernel,
        out_type=(jax.ShapeDtypeStruct((NPAD,), _f32),
                  jax.ShapeDtypeStruct((NPAD,), _f32),
                  jax.ShapeDtypeStruct((NTILE, NPAD), _f32)),
        mesh=_MESH,
        name="sc_p1f" if first else "sc_p1r",
        compiler_params=_SC_PARAMS,
        scratch_types=[
            pltpu.VMEM((NPAD,), _f32),
            pltpu.VMEM((NPAD,), _f32),
            pltpu.VMEM((_RCPG, CHUNK), _i32),
            pltpu.VMEM((_RCPG, CHUNK), _i32),
            pltpu.VMEM((RPT,), _f32),
            pltpu.VMEM((RPT,), _f32),
            pltpu.VMEM((RPT,), _f32),
        ],
    )
    def p1(u_hbm, src_hbm, dstef_hbm, dis_hbm, t0_hbm, tx_hbm, uo_hbm, stage,
           utab, accl, sbuf, dbuf, abuf, tbuf, disb):
        cid = lax.axis_index("c")
        sid = lax.axis_index("s")
        on0 = cid == 0

        @pl.when(on0)
        def _main():
            _zero_vec(accl, NPAD)
            pltpu.sync_copy(u_hbm, utab)
            def group(g, carry):
                base = sid * CPT + g * _RCPG
                pltpu.sync_copy(src_hbm.at[pl.ds(base, _RCPG)], sbuf)
                pltpu.sync_copy(dstef_hbm.at[pl.ds(base, _RCPG)], dbuf)
                for c in range(_RCPG):
                    for v in range(VPC):
                        sl = pl.ds(v * LANES, LANES)
                        val = plsc.load_gather(utab, [sbuf[c, sl]])
                        plsc.addupdate_scatter(accl, [dbuf[c, sl]], val)
                return carry
            lax.fori_loop(0, _RGPT, group, 0)
            pltpu.sync_copy(accl, stage.at[sid])

        plsc.subcore_barrier()

        @pl.when(on0)
        def _drain():
            r0 = sid * RPT
            _reduce_stage(stage, abuf, tbuf, r0)
            pltpu.sync_copy(dis_hbm.at[pl.ds(r0, RPT)], disb)
            if not first:
                pltpu.sync_copy(t0_hbm.at[pl.ds(r0, RPT)], tbuf)
            def drow(r, carry):
                sl = pl.ds(r * LANES, LANES)
                a = abuf[sl]
                d = disb[sl]
                if first:
                    t = -(a * d)
                else:
                    t = (a * d) * (-2.0) - tbuf[sl]
                abuf[sl] = t
                disb[sl] = t * d
                return carry
            lax.fori_loop(0, RPT // LANES, drow, 0)
            pltpu.sync_copy(abuf, tx_hbm.at[pl.ds(r0, RPT)])
            pltpu.sync_copy(disb, uo_hbm.at[pl.ds(r0, RPT)])

    return p1


_P1_FIRST = _make_p1(True)
_P1_REST = _make_p1(False)


def _make_p32(first):
    @functools.partial(
        pl.kernel,
        out_type=(jax.ShapeDtypeStruct((NQ * NPAD, QCOL), _f32),
                  jax.ShapeDtypeStruct((NQ * NPAD, QCOL), _f32)),
        mesh=_MESH,
        name="sc_p32f" if first else "sc_p32r",
        compiler_params=_SC_PARAMS,
        scratch_types=[
            pltpu.VMEM((2, CPG, CHUNK), _i32),
            pltpu.VMEM((2, CPG, CHUNK), _i32),
            pltpu.VMEM((2, CPG, CHUNK, QCOL), _f32),
            pltpu.VMEM((DR32, QCOL), _f32),
            pltpu.VMEM((DR32, QCOL), _f32),
            pltpu.VMEM((DR32, QCOL), _f32),
            pltpu.VMEM((DR32, QCOL), _f32),
            pltpu.VMEM((DR32, QCOL), _f32),
            pltpu.VMEM_SHARED((NPAD, QCOL), _f32),
            pltpu.SemaphoreType.DMA,
            pltpu.SemaphoreType.DMA,
            pltpu.SemaphoreType.DMA,
        ],
    )
    def p32(u_hbm, srcef_hbm, dstef_hbm, disb_hbm, t0_hbm, tx_hbm, uo_hbm,
            sbuf, dbuf, rbuf, abuf, t0b, dbb, txb, uob, acc,
            sem_i, sem_g, sem_s):
        cid = lax.axis_index("c")
        sid = lax.axis_index("s")

        def idx_load(qq, g, sl, sync):
            sbase = qq * NCHUNK + sid * CPT + g * CPG
            dbase = sid * CPT + g * CPG
            if sync:
                pltpu.sync_copy(srcef_hbm.at[pl.ds(sbase, CPG)], sbuf.at[sl])
                pltpu.sync_copy(dstef_hbm.at[pl.ds(dbase, CPG)], dbuf.at[sl])
            else:
                pltpu.async_copy(srcef_hbm.at[pl.ds(sbase, CPG)],
                                 sbuf.at[sl], sem_i)
                pltpu.async_copy(dstef_hbm.at[pl.ds(dbase, CPG)],
                                 dbuf.at[sl], sem_i)

        def wait_idx():
            for _ in range(2):
                pltpu.make_async_copy(
                    srcef_hbm.at[pl.ds(0, CPG)], sbuf.at[0], sem_i).wait()

        def fire_gathers(sl):
            for c in range(CPG):
                pltpu.async_copy(u_hbm.at[sbuf.at[sl, c]],
                                 rbuf.at[sl, c], sem_g)

        def wait_gathers():
            for _ in range(CPG):
                pltpu.make_async_copy(
                    u_hbm.at[sbuf.at[0, 0]], rbuf.at[0, 0], sem_g).wait()

        def fire_scatters(sl):
            for c in range(CPG):
                pltpu.async_copy(rbuf.at[sl, c], acc.at[dbuf.at[sl, c]],
                                 sem_s, add=True)

        def wait_scatters():
            for _ in range(CPG):
                pltpu.make_async_copy(
                    rbuf.at[0, 0], acc.at[dbuf.at[0, 0]], sem_s).wait()

        for q in range(NCORE):
            qq = cid * NCORE + q

            _zero_fill(abuf, DR32, QCOL)
            for it in range(RPT // DR32):
                pltpu.sync_copy(
                    abuf, acc.at[pl.ds(sid * RPT + it * DR32, DR32)])

            plsc.subcore_barrier()

            def ring_iter(g, slot, wait_sc):
                if wait_sc:
                    wait_scatters()
                idx_load(qq, g, slot, sync=False)
                wait_gathers()
                fire_scatters(1 - slot)
                wait_idx()
                fire_gathers(slot)

            idx_load(qq, 0, 0, sync=True)
            fire_gathers(0)
            ring_iter(1, 1, False)
            ring_iter(2, 0, True)
            def pair(j, carry):
                g = 2 * j + 3
                ring_iter(g, 1, True)
                ring_iter(g + 1, 0, True)
                return carry
            lax.fori_loop(0, (GPT - 3) // 2, pair, 0)
            wait_gathers()
            fire_scatters((GPT - 1) % 2)
            wait_scatters()
            wait_scatters()

            plsc.subcore_barrier()

            def drain(it, carry):
                rloc = sid * RPT + it * DR32
                rglob = qq * NPAD + rloc
                pltpu.sync_copy(acc.at[pl.ds(rloc, DR32)], abuf)
                pltpu.sync_copy(disb_hbm.at[pl.ds(rloc, DR32)], dbb)
                if not first:
                    pltpu.sync_copy(t0_hbm.at[pl.ds(rglob, DR32)], t0b)
                def drow(r, carry2):
                    sl = pl.ds(0, LANES)
                    a = abuf[r, sl]
                    d = dbb[r, sl]
                    if first:
                        t = -(a * d)
                    else:
                        t = (a * d) * (-2.0) - t0b[r, sl]
                    txb[r, sl] = t
                    uob[r, sl] = t * d
                    return carry2
                lax.fori_loop(0, DR32, drow, 0)
                pltpu.sync_copy(txb, tx_hbm.at[pl.ds(rglob, DR32)])
                pltpu.sync_copy(uob, uo_hbm.at[pl.ds(rglob, DR32)])
                return carry
            lax.fori_loop(0, RPT // DR32, drain, 0)

            if q == 0:
                plsc.subcore_barrier()

    return p32


_P32_FIRST = _make_p32(True)
_P32_REST = _make_p32(False)


_BLK = 512
_NB = NPAD // _BLK


def _tc_mid_body(tx_ref, w_ref, b_ref, dis_ref, *out_refs):
    h = jnp.dot(tx_ref[...], w_ref[...],
                preferred_element_type=_f32,
                precision=lax.Precision.HIGHEST)
    h = jnp.maximum(h + b_ref[...], 0.0)
    u = h * dis_ref[...]
    for qq in range(NQ):
        out_refs[qq][...] = h[:, qq * QCOL:(qq + 1) * QCOL]
        out_refs[NQ + qq][...] = u[:, qq * QCOL:(qq + 1) * QCOL]


def _tc_mid(txs, w_r, b, dis2d, kdim):
    outs = pl.pallas_call(
        _tc_mid_body,
        grid=(_NB,),
        in_specs=[
            pl.BlockSpec((_BLK, kdim), lambda i: (i, 0)),
            pl.BlockSpec((kdim, 64), lambda i: (0, 0)),
            pl.BlockSpec((1, 64), lambda i: (0, 0)),
            pl.BlockSpec((_BLK, 1), lambda i: (i, 0)),
        ],
        out_specs=[pl.BlockSpec((_BLK, QCOL), lambda i: (i, 0))] * (2 * NQ),
        out_shape=[jax.ShapeDtypeStruct((NPAD, QCOL), _f32)] * (2 * NQ),
    )(txs, w_r, b.reshape(1, 64), dis2d)
    h_tbl = jnp.concatenate(outs[:NQ], axis=0)
    u_tbl = jnp.concatenate(outs[NQ:], axis=0)
    return h_tbl, u_tbl


def _tc_final_body(tx_ref, w_ref, b_ref, fcw_ref, fcb_ref, o_ref):
    h = jnp.dot(tx_ref[...], w_ref[...],
                preferred_element_type=_f32,
                precision=lax.Precision.HIGHEST)
    h = jnp.maximum(h + b_ref[...], 0.0)
    o_ref[...] = jnp.dot(h, fcw_ref[...],
                         preferred_element_type=_f32,
                         precision=lax.Precision.HIGHEST) + fcb_ref[...]


def _tc_final(txs, w_r, b, fcw, fcb):
    return pl.pallas_call(
        _tc_final_body,
        grid=(_NB,),
        in_specs=[
            pl.BlockSpec((_BLK, 320), lambda i: (i, 0)),
            pl.BlockSpec((320, 64), lambda i: (0, 0)),
            pl.BlockSpec((1, 64), lambda i: (0, 0)),
            pl.BlockSpec((64, 1), lambda i: (0, 0)),
            pl.BlockSpec((1, 1), lambda i: (0, 0)),
        ],
        out_specs=pl.BlockSpec((_BLK, 1), lambda i: (i, 0)),
        out_shape=jax.ShapeDtypeStruct((NPAD, 1), _f32),
    )(txs, w_r, b.reshape(1, 64), fcw, fcb.reshape(1, 1))


def _cat32(parts):
    cols = []
    for t in parts:
        for qq in range(NQ):
            cols.append(lax.slice_in_dim(t, qq * NPAD, (qq + 1) * NPAD,
                                         axis=0))
    return jnp.concatenate(cols, axis=1)


def _layer32(u_tbl, tx0_tbl, srcef, dstef, disb, dis2d, w, b,
             final_args=None):
    t1, v1 = _P32_FIRST(u_tbl, srcef, dstef, disb, tx0_tbl)
    t2, v2 = _P32_REST(v1, srcef, dstef, disb, tx0_tbl)
    t3, v3 = _P32_REST(v2, srcef, dstef, disb, t1)
    t4, _ = _P32_REST(v3, srcef, dstef, disb, t2)
    txcat = _cat32([tx0_tbl, t1, t2, t3, t4])
    w_r = w.reshape(KORD * 64, 64)
    if final_args is None:
        return _tc_mid(txcat, w_r, b, dis2d, 320)
    fcw, fcb = final_args
    return _tc_final(txcat, w_r, b, fcw, fcb)


def kernel(x, edge_index, W1, b1, W2, b2, W3, b3, fcw, fcb):
    src = jnp.pad(edge_index[0], (0, EPAD - NEDGE)).reshape(NCHUNK, CHUNK)
    dst = jnp.pad(edge_index[1], (0, EPAD - NEDGE)).reshape(NCHUNK, CHUNK)
    x_pad = jnp.pad(x[:, 0], (0, NPAD - NNODE))

    deg, _ = _deg_kernel(src, dst)
    dis = jnp.where(deg > 0, lax.rsqrt(jnp.maximum(deg, 1e-30)), 0.0)
    dis2d = dis[:, None]
    disb = jnp.broadcast_to(dis2d, (NPAD, QCOL))

    srcef, dstef, u0 = _prep_kernel(src, dst, dis, x_pad)

    t1, v1, _ = _P1_FIRST(u0, src, dstef, dis, u0)
    t2, v2, _ = _P1_REST(v1, src, dstef, dis, x_pad)
    t3, v3, _ = _P1_REST(v2, src, dstef, dis, t1)
    t4, _, _ = _P1_REST(v3, src, dstef, dis, t2)
    txs1 = jnp.stack([x_pad, t1, t2, t3, t4], axis=1)

    h1f, u1f = _tc_mid(txs1, W1.reshape(KORD, 64), b1, dis2d, KORD)

    h2f, u2f = _layer32(u1f, h1f, srcef, dstef, disb, dis2d, W2, b2)

    y = _layer32(u2f, h2f, srcef, dstef, disb, dis2d, W3, b3,
                 final_args=(fcw, fcb))
    return y[:NNODE]


# speedup vs baseline: 1.3861x; 1.3861x over previous
import functools

import jax
import jax.numpy as jnp
from jax import lax
from jax.experimental import pallas as pl
from jax.experimental.pallas import tpu as pltpu
from jax.experimental.pallas import tpu_sc as plsc

NNODE = 50000
KORD = 5
NEDGE = 800000

NTILE = 16
NCORE = 2
LANES = 16

NPAD = 50176
DEAD = NNODE
CHUNK = 128
EPAD = 802816
NCHUNK = EPAD // CHUNK
QCOL = 16
NQ = 4
CPT = NCHUNK // NTILE
CPG = 8
GPT = CPT // CPG
RPT = NPAD // NTILE
DR32 = 224
VPC = CHUNK // LANES

_MESH = plsc.VectorSubcoreMesh(
    core_axis_name="c", subcore_axis_name="s",
    num_cores=NCORE, num_subcores=NTILE)
_SC_PARAMS = pltpu.CompilerParams(
    needs_layout_passes=False, use_tc_tiling_on_sc=False)

_f32 = jnp.float32
_i32 = jnp.int32


def _zero_fill(ref, nrows, ncols):
    z = jnp.zeros((LANES,), _f32)
    def row(r, carry):
        for v in range(ncols // LANES):
            ref[r, pl.ds(v * LANES, LANES)] = z
        return carry
    lax.fori_loop(0, nrows, row, 0)


_RCPG = 28
_RGPT = CPT // _RCPG


def _zero_vec(ref, nwords):
    zero = jnp.zeros((LANES,), _f32)
    def zrow(r, carry):
        for u in range(8):
            ref[pl.ds((r * 8 + u) * LANES, LANES)] = zero
        return carry
    lax.fori_loop(0, nwords // (8 * LANES), zrow, 0)


def _reduce_stage(stage, abuf, tbuf, r0):
    pltpu.sync_copy(stage.at[0, pl.ds(r0, RPT)], abuf)
    for t in range(1, NTILE):
        pltpu.sync_copy(stage.at[t, pl.ds(r0, RPT)], tbuf)
        def arow(r, carry):
            sl = pl.ds(r * LANES, LANES)
            abuf[sl] = abuf[sl] + tbuf[sl]
            return carry
        lax.fori_loop(0, RPT // LANES, arow, 0)


@functools.partial(
    pl.kernel,
    out_type=(jax.ShapeDtypeStruct((NPAD,), _f32),
              jax.ShapeDtypeStruct((NTILE, NPAD), _f32)),
    mesh=_MESH,
    name="sc_deg",
    compiler_params=_SC_PARAMS,
    scratch_types=[
        pltpu.VMEM((_RCPG, CHUNK), _i32),
        pltpu.VMEM((_RCPG, CHUNK), _i32),
        pltpu.VMEM((NPAD,), _f32),
        pltpu.VMEM((RPT,), _f32),
        pltpu.VMEM((RPT,), _f32),
    ],
)
def _deg_kernel(src_hbm, dst_hbm, deg_hbm, stage,
                sbuf, dbuf, accl, abuf, tbuf):
    cid = lax.axis_index("c")
    sid = lax.axis_index("s")
    on0 = cid == 0

    @pl.when(on0)
    def _main():
        _zero_vec(accl, NPAD)
        ones = jnp.ones((LANES,), _f32)
        def group(g, carry):
            base = sid * CPT + g * _RCPG
            pltpu.sync_copy(src_hbm.at[pl.ds(base, _RCPG)], sbuf)
            pltpu.sync_copy(dst_hbm.at[pl.ds(base, _RCPG)], dbuf)
            for c in range(_RCPG):
                for v in range(VPC):
                    sl = pl.ds(v * LANES, LANES)
                    s = sbuf[c, sl]
                    d = dbuf[c, sl]
                    plsc.addupdate_scatter(
                        accl, [jnp.where(s != d, s, DEAD)], ones)
            return carry
        lax.fori_loop(0, _RGPT, group, 0)
        pltpu.sync_copy(accl, stage.at[sid])

    plsc.subcore_barrier()

    @pl.when(on0)
    def _drain():
        r0 = sid * RPT
        _reduce_stage(stage, abuf, tbuf, r0)
        pltpu.sync_copy(abuf, deg_hbm.at[pl.ds(r0, RPT)])


_B_NW = 28
_B_CPW = NCHUNK // _B_NW
_B_CPG = 8
_B_GPW = _B_CPW // _B_CPG
_B_NPW = NPAD // (NCORE * NTILE)

@functools.partial(
    pl.kernel,
    out_type=(jax.ShapeDtypeStruct((NQ * NCHUNK, CHUNK), _i32),
              jax.ShapeDtypeStruct((NCHUNK, CHUNK), _i32),
              jax.ShapeDtypeStruct((NPAD,), _f32)),
    mesh=_MESH,
    name="sc_prep",
    compiler_params=_SC_PARAMS,
    scratch_types=[
        pltpu.VMEM((_B_CPG, CHUNK), _i32),
        pltpu.VMEM((_B_CPG, CHUNK), _i32),
        pltpu.VMEM((_B_CPG, CHUNK), _i32),
        pltpu.VMEM((_B_NPW,), _f32),
        pltpu.VMEM((_B_NPW,), _f32),
        pltpu.VMEM((_B_NPW,), _f32),
    ],
)
def _prep_kernel(src_hbm, dst_hbm, dis_hbm, x_hbm,
                 srcef_hbm, dstef_hbm, u0_hbm,
                 sbuf, dbuf, hbuf, xbuf, disbuf, ubuf):
    cid = lax.axis_index("c")
    sid = lax.axis_index("s")
    wid = sid * NCORE + cid

    @pl.when(wid < _B_NW)
    def _edges():
        def group(g, carry):
            base = wid * _B_CPW + g * _B_CPG
            pltpu.sync_copy(src_hbm.at[pl.ds(base, _B_CPG)], sbuf)
            pltpu.sync_copy(dst_hbm.at[pl.ds(base, _B_CPG)], dbuf)
            for qq in range(NQ):
                for c in range(_B_CPG):
                    for v in range(VPC):
                        sl = pl.ds(v * LANES, LANES)
                        hbuf[c, sl] = sbuf[c, sl] * NQ + qq
                pltpu.sync_copy(
                    hbuf, srcef_hbm.at[pl.ds(qq * NCHUNK + base, _B_CPG)])
            for c in range(_B_CPG):
                for v in range(VPC):
                    sl = pl.ds(v * LANES, LANES)
                    s = sbuf[c, sl]
                    d = dbuf[c, sl]
                    dbuf[c, sl] = jnp.where(s != d, d, DEAD)
            pltpu.sync_copy(dbuf, dstef_hbm.at[pl.ds(base, _B_CPG)])
            return carry
        lax.fori_loop(0, _B_GPW, group, 0)

    nbase = wid * _B_NPW
    pltpu.sync_copy(x_hbm.at[pl.ds(nbase, _B_NPW)], xbuf)
    pltpu.sync_copy(dis_hbm.at[pl.ds(nbase, _B_NPW)], disbuf)
    def nrow(r, carry):
        sl = pl.ds(r * LANES, LANES)
        ubuf[sl] = xbuf[sl] * disbuf[sl]
        return carry
    lax.fori_loop(0, _B_NPW // LANES, nrow, 0)
    pltpu.sync_copy(ubuf, u0_hbm.at[pl.ds(nbase, _B_NPW)])


def _make_p1(first):
    @functools.partial(
        pl.kernel,
        out_type=(jax.ShapeDtypeStruct((NPAD,), _f32),
                  jax.ShapeDtypeStruct((NPAD,), _f32),
                  jax.ShapeDtypeStruct((NTILE, NPAD), _f32)),
        mesh=_MESH,
        name="sc_p1f" if first else "sc_p1r",
        compiler_params=_SC_PARAMS,
        scratch_types=[
            pltpu.VMEM((NPAD,), _f32),
            pltpu.VMEM((NPAD,), _f32),
            pltpu.VMEM((_RCPG, CHUNK), _i32),
            pltpu.VMEM((_RCPG, CHUNK), _i32),
            pltpu.VMEM((RPT,), _f32),
            pltpu.VMEM((RPT,), _f32),
            pltpu.VMEM((RPT,), _f32),
        ],
    )
    def p1(u_hbm, src_hbm, dstef_hbm, dis_hbm, t0_hbm, tx_hbm, uo_hbm, stage,
           utab, accl, sbuf, dbuf, abuf, tbuf, disb):
        cid = lax.axis_index("c")
        sid = lax.axis_index("s")
        on0 = cid == 0

        @pl.when(on0)
        def _main():
            _zero_vec(accl, NPAD)
            pltpu.sync_copy(u_hbm, utab)
            def group(g, carry):
                base = sid * CPT + g * _RCPG
                pltpu.sync_copy(src_hbm.at[pl.ds(base, _RCPG)], sbuf)
                pltpu.sync_copy(dstef_hbm.at[pl.ds(base, _RCPG)], dbuf)
                for c in range(_RCPG):
                    for v in range(VPC):
                        sl = pl.ds(v * LANES, LANES)
                        val = plsc.load_gather(utab, [sbuf[c, sl]])
                        plsc.addupdate_scatter(accl, [dbuf[c, sl]], val)
                return carry
            lax.fori_loop(0, _RGPT, group, 0)
            pltpu.sync_copy(accl, stage.at[sid])

        plsc.subcore_barrier()

        @pl.when(on0)
        def _drain():
            r0 = sid * RPT
            _reduce_stage(stage, abuf, tbuf, r0)
            pltpu.sync_copy(dis_hbm.at[pl.ds(r0, RPT)], disb)
            if not first:
                pltpu.sync_copy(t0_hbm.at[pl.ds(r0, RPT)], tbuf)
            def drow(r, carry):
                sl = pl.ds(r * LANES, LANES)
                a = abuf[sl]
                d = disb[sl]
                if first:
                    t = -(a * d)
                else:
                    t = (a * d) * (-2.0) - tbuf[sl]
                abuf[sl] = t
                disb[sl] = t * d
                return carry
            lax.fori_loop(0, RPT // LANES, drow, 0)
            pltpu.sync_copy(abuf, tx_hbm.at[pl.ds(r0, RPT)])
            pltpu.sync_copy(disb, uo_hbm.at[pl.ds(r0, RPT)])

    return p1


_P1_FIRST = _make_p1(True)
_P1_REST = _make_p1(False)


def _make_p32(first):
    @functools.partial(
        pl.kernel,
        out_type=(jax.ShapeDtypeStruct((NPAD, 64), _f32),
                  jax.ShapeDtypeStruct((NPAD, 64), _f32)),
        mesh=_MESH,
        name="sc_p32f" if first else "sc_p32r",
        compiler_params=_SC_PARAMS,
        scratch_types=[
            pltpu.VMEM((2, CPG, CHUNK), _i32),
            pltpu.VMEM((2, CPG, CHUNK), _i32),
            pltpu.VMEM((2, CPG, CHUNK, QCOL), _f32),
            pltpu.VMEM((DR32, QCOL), _f32),
            pltpu.VMEM((DR32, QCOL), _f32),
            pltpu.VMEM((DR32, QCOL), _f32),
            pltpu.VMEM((DR32, QCOL), _f32),
            pltpu.VMEM((DR32, QCOL), _f32),
            pltpu.VMEM_SHARED((NPAD, QCOL), _f32),
            pltpu.SemaphoreType.DMA,
            pltpu.SemaphoreType.DMA,
            pltpu.SemaphoreType.DMA,
        ],
    )
    def p32(u_hbm, srcef_hbm, dstef_hbm, disb_hbm, t0_hbm, tx_hbm, uo_hbm,
            sbuf, dbuf, rbuf, abuf, t0b, dbb, txb, uob, acc,
            sem_i, sem_g, sem_s):
        cid = lax.axis_index("c")
        sid = lax.axis_index("s")

        def idx_load(qq, g, sl, sync):
            sbase = qq * NCHUNK + sid * CPT + g * CPG
            dbase = sid * CPT + g * CPG
            if sync:
                pltpu.sync_copy(srcef_hbm.at[pl.ds(sbase, CPG)], sbuf.at[sl])
                pltpu.sync_copy(dstef_hbm.at[pl.ds(dbase, CPG)], dbuf.at[sl])
            else:
                pltpu.async_copy(srcef_hbm.at[pl.ds(sbase, CPG)],
                                 sbuf.at[sl], sem_i)
                pltpu.async_copy(dstef_hbm.at[pl.ds(dbase, CPG)],
                                 dbuf.at[sl], sem_i)

        def wait_idx():
            for _ in range(2):
                pltpu.make_async_copy(
                    srcef_hbm.at[pl.ds(0, CPG)], sbuf.at[0], sem_i).wait()

        def fire_gathers(sl):
            for c in range(CPG):
                pltpu.async_copy(u_hbm.at[sbuf.at[sl, c]],
                                 rbuf.at[sl, c], sem_g)

        def wait_gathers():
            for _ in range(CPG):
                pltpu.make_async_copy(
                    u_hbm.at[sbuf.at[0, 0]], rbuf.at[0, 0], sem_g).wait()

        def fire_scatters(sl):
            for c in range(CPG):
                pltpu.async_copy(rbuf.at[sl, c], acc.at[dbuf.at[sl, c]],
                                 sem_s, add=True)

        def wait_scatters():
            for _ in range(CPG):
                pltpu.make_async_copy(
                    rbuf.at[0, 0], acc.at[dbuf.at[0, 0]], sem_s).wait()

        for q in range(NCORE):
            qq = cid * NCORE + q

            _zero_fill(abuf, DR32, QCOL)
            for it in range(RPT // DR32):
                pltpu.sync_copy(
                    abuf, acc.at[pl.ds(sid * RPT + it * DR32, DR32)])

            plsc.subcore_barrier()

            def ring_iter(g, slot, wait_sc):
                if wait_sc:
                    wait_scatters()
                idx_load(qq, g, slot, sync=False)
                wait_gathers()
                fire_scatters(1 - slot)
                wait_idx()
                fire_gathers(slot)

            idx_load(qq, 0, 0, sync=True)
            fire_gathers(0)
            ring_iter(1, 1, False)
            ring_iter(2, 0, True)
            def pair(j, carry):
                g = 2 * j + 3
                ring_iter(g, 1, True)
                ring_iter(g + 1, 0, True)
                return carry
            lax.fori_loop(0, (GPT - 3) // 2, pair, 0)
            wait_gathers()
            fire_scatters((GPT - 1) % 2)
            wait_scatters()
            wait_scatters()

            plsc.subcore_barrier()

            def drain(it, carry):
                rloc = sid * RPT + it * DR32
                csl = pl.ds(qq * QCOL, QCOL)
                pltpu.sync_copy(acc.at[pl.ds(rloc, DR32)], abuf)
                pltpu.sync_copy(disb_hbm.at[pl.ds(rloc, DR32)], dbb)
                if not first:
                    pltpu.sync_copy(t0_hbm.at[pl.ds(rloc, DR32), csl], t0b)
                def drow(r, carry2):
                    sl = pl.ds(0, LANES)
                    a = abuf[r, sl]
                    d = dbb[r, sl]
                    if first:
                        t = -(a * d)
                    else:
                        t = (a * d) * (-2.0) - t0b[r, sl]
                    txb[r, sl] = t
                    uob[r, sl] = t * d
                    return carry2
                lax.fori_loop(0, DR32, drow, 0)
                pltpu.sync_copy(txb, tx_hbm.at[pl.ds(rloc, DR32), csl])
                pltpu.sync_copy(uob, uo_hbm.at[pl.ds(rloc, DR32), csl])
                return carry
            lax.fori_loop(0, RPT // DR32, drain, 0)

            if q == 0:
                plsc.subcore_barrier()

    return p32


_P32_FIRST = _make_p32(True)
_P32_REST = _make_p32(False)


_BLK = 512
_NB = NPAD // _BLK


def _make_tc_mid(nk):
    def body(*refs):
        tx_refs = refs[:nk]
        w_ref, b_ref, dis_ref, h_ref, u_ref = refs[nk:]
        acc = b_ref[...].astype(_f32) * jnp.ones((_BLK, 1), _f32)
        for k in range(nk):
            acc = acc + jnp.dot(tx_refs[k][...],
                                w_ref[pl.ds(k * 64, 64), :],
                                preferred_element_type=_f32,
                                precision=lax.Precision.HIGHEST)
        h = jnp.maximum(acc, 0.0)
        h_ref[...] = h
        u_ref[...] = h * dis_ref[...]
    return body


def _tc_mid(txs, w_r, b, dis2d):
    nk = len(txs)
    h, u = pl.pallas_call(
        _make_tc_mid(nk),
        grid=(_NB,),
        in_specs=(
            [pl.BlockSpec((_BLK, 64), lambda i: (i, 0))] * nk
            + [pl.BlockSpec((nk * 64, 64), lambda i: (0, 0)),
               pl.BlockSpec((1, 64), lambda i: (0, 0)),
               pl.BlockSpec((_BLK, 1), lambda i: (i, 0))]
        ),
        out_specs=[pl.BlockSpec((_BLK, 64), lambda i: (i, 0))] * 2,
        out_shape=[jax.ShapeDtypeStruct((NPAD, 64), _f32)] * 2,
    )(*txs, w_r, b.reshape(1, 64), dis2d)
    return h, u


def _tc1_body(tx_ref, w_ref, b_ref, dis_ref, h_ref, u_ref):
    h = jnp.dot(tx_ref[...], w_ref[...],
                preferred_element_type=_f32,
                precision=lax.Precision.HIGHEST)
    h = jnp.maximum(h + b_ref[...], 0.0)
    h_ref[...] = h
    u_ref[...] = h * dis_ref[...]


def _tc1(txs1, w_r, b, dis2d):
    h, u = pl.pallas_call(
        _tc1_body,
        grid=(_NB,),
        in_specs=[
            pl.BlockSpec((_BLK, KORD), lambda i: (i, 0)),
            pl.BlockSpec((KORD, 64), lambda i: (0, 0)),
            pl.BlockSpec((1, 64), lambda i: (0, 0)),
            pl.BlockSpec((_BLK, 1), lambda i: (i, 0)),
        ],
        out_specs=[pl.BlockSpec((_BLK, 64), lambda i: (i, 0))] * 2,
        out_shape=[jax.ShapeDtypeStruct((NPAD, 64), _f32)] * 2,
    )(txs1, w_r, b.reshape(1, 64), dis2d)
    return h, u


def _make_tc_final(nk):
    def body(*refs):
        tx_refs = refs[:nk]
        w_ref, b_ref, fcw_ref, fcb_ref, o_ref = refs[nk:]
        acc = b_ref[...].astype(_f32) * jnp.ones((_BLK, 1), _f32)
        for k in range(nk):
            acc = acc + jnp.dot(tx_refs[k][...],
                                w_ref[pl.ds(k * 64, 64), :],
                                preferred_element_type=_f32,
                                precision=lax.Precision.HIGHEST)
        h = jnp.maximum(acc, 0.0)
        o_ref[...] = jnp.dot(h, fcw_ref[...],
                             preferred_element_type=_f32,
                             precision=lax.Precision.HIGHEST) + fcb_ref[...]
    return body


def _tc_final(txs, w_r, b, fcw, fcb):
    nk = len(txs)
    return pl.pallas_call(
        _make_tc_final(nk),
        grid=(_NB,),
        in_specs=(
            [pl.BlockSpec((_BLK, 64), lambda i: (i, 0))] * nk
            + [pl.BlockSpec((nk * 64, 64), lambda i: (0, 0)),
               pl.BlockSpec((1, 64), lambda i: (0, 0)),
               pl.BlockSpec((64, 1), lambda i: (0, 0)),
               pl.BlockSpec((1, 1), lambda i: (0, 0))]
        ),
        out_specs=pl.BlockSpec((_BLK, 1), lambda i: (i, 0)),
        out_shape=jax.ShapeDtypeStruct((NPAD, 1), _f32),
    )(*txs, w_r, b.reshape(1, 64), fcw, fcb.reshape(1, 1))


def _layer32(u_tbl, tx0_tbl, srcef, dstef, disb, dis2d, w, b,
             final_args=None):
    qview = lambda t: t.reshape(NQ * NPAD, QCOL)
    t1, v1 = _P32_FIRST(qview(u_tbl), srcef, dstef, disb, tx0_tbl)
    t2, v2 = _P32_REST(qview(v1), srcef, dstef, disb, tx0_tbl)
    t3, v3 = _P32_REST(qview(v2), srcef, dstef, disb, t1)
    t4, _ = _P32_REST(qview(v3), srcef, dstef, disb, t2)
    txs = [tx0_tbl, t1, t2, t3, t4]
    w_r = w.reshape(KORD * 64, 64)
    if final_args is None:
        return _tc_mid(txs, w_r, b, dis2d)
    fcw, fcb = final_args
    return _tc_final(txs, w_r, b, fcw, fcb)


def kernel(x, edge_index, W1, b1, W2, b2, W3, b3, fcw, fcb):
    src = jnp.pad(edge_index[0], (0, EPAD - NEDGE)).reshape(NCHUNK, CHUNK)
    dst = jnp.pad(edge_index[1], (0, EPAD - NEDGE)).reshape(NCHUNK, CHUNK)
    x_pad = jnp.pad(x[:, 0], (0, NPAD - NNODE))

    deg, _ = _deg_kernel(src, dst)
    dis = jnp.where(deg > 0, lax.rsqrt(jnp.maximum(deg, 1e-30)), 0.0)
    dis2d = dis[:, None]
    disb = jnp.broadcast_to(dis2d, (NPAD, QCOL))

    srcef, dstef, u0 = _prep_kernel(src, dst, dis, x_pad)

    t1, v1, _ = _P1_FIRST(u0, src, dstef, dis, u0)
    t2, v2, _ = _P1_REST(v1, src, dstef, dis, x_pad)
    t3, v3, _ = _P1_REST(v2, src, dstef, dis, t1)
    t4, _, _ = _P1_REST(v3, src, dstef, dis, t2)
    txs1 = jnp.stack([x_pad, t1, t2, t3, t4], axis=1)

    h1, u1 = _tc1(txs1, W1.reshape(KORD, 64), b1, dis2d)

    h2, u2 = _layer32(u1, h1, srcef, dstef, disb, dis2d, W2, b2)

    y = _layer32(u2, h2, srcef, dstef, disb, dis2d, W3, b3,
                 final_args=(fcw, fcb))
    return y[:NNODE]


# speedup vs baseline: 1.5280x; 1.1024x over previous
import functools

import jax
import jax.numpy as jnp
from jax import lax
from jax.experimental import pallas as pl
from jax.experimental.pallas import tpu as pltpu
from jax.experimental.pallas import tpu_sc as plsc

NNODE = 50000
KORD = 5
NEDGE = 800000

NTILE = 16
NCORE = 2
LANES = 16

NPAD = 50176
DEAD = NNODE
CHUNK = 128
EPAD = 802816
NCHUNK = EPAD // CHUNK
QCOL = 16
NQ = 4
CPT = NCHUNK // NTILE
CPG = 4
GPT = CPT // CPG
RPT = NPAD // NTILE
DR32 = 224
VPC = CHUNK // LANES

_MESH = plsc.VectorSubcoreMesh(
    core_axis_name="c", subcore_axis_name="s",
    num_cores=NCORE, num_subcores=NTILE)
_SC_PARAMS = pltpu.CompilerParams(
    needs_layout_passes=False, use_tc_tiling_on_sc=False)

_f32 = jnp.float32
_i32 = jnp.int32


def _zero_fill(ref, nrows, ncols):
    z = jnp.zeros((LANES,), _f32)
    def row(r, carry):
        for v in range(ncols // LANES):
            ref[r, pl.ds(v * LANES, LANES)] = z
        return carry
    lax.fori_loop(0, nrows, row, 0)


_RCPG = 28
_RGPT = CPT // _RCPG


def _zero_vec(ref, nwords):
    zero = jnp.zeros((LANES,), _f32)
    def zrow(r, carry):
        for u in range(8):
            ref[pl.ds((r * 8 + u) * LANES, LANES)] = zero
        return carry
    lax.fori_loop(0, nwords // (8 * LANES), zrow, 0)


def _reduce_stage(stage, abuf, tbuf, r0):
    pltpu.sync_copy(stage.at[0, pl.ds(r0, RPT)], abuf)
    for t in range(1, NTILE):
        pltpu.sync_copy(stage.at[t, pl.ds(r0, RPT)], tbuf)
        def arow(r, carry):
            sl = pl.ds(r * LANES, LANES)
            abuf[sl] = abuf[sl] + tbuf[sl]
            return carry
        lax.fori_loop(0, RPT // LANES, arow, 0)


@functools.partial(
    pl.kernel,
    out_type=(jax.ShapeDtypeStruct((NPAD,), _f32),
              jax.ShapeDtypeStruct((NTILE, NPAD), _f32)),
    mesh=_MESH,
    name="sc_deg",
    compiler_params=_SC_PARAMS,
    scratch_types=[
        pltpu.VMEM((_RCPG, CHUNK), _i32),
        pltpu.VMEM((_RCPG, CHUNK), _i32),
        pltpu.VMEM((NPAD,), _f32),
        pltpu.VMEM((RPT,), _f32),
        pltpu.VMEM((RPT,), _f32),
    ],
)
def _deg_kernel(src_hbm, dst_hbm, deg_hbm, stage,
                sbuf, dbuf, accl, abuf, tbuf):
    cid = lax.axis_index("c")
    sid = lax.axis_index("s")
    on0 = cid == 0

    @pl.when(on0)
    def _main():
        _zero_vec(accl, NPAD)
        ones = jnp.ones((LANES,), _f32)
        def group(g, carry):
            base = sid * CPT + g * _RCPG
            pltpu.sync_copy(src_hbm.at[pl.ds(base, _RCPG)], sbuf)
            pltpu.sync_copy(dst_hbm.at[pl.ds(base, _RCPG)], dbuf)
            for c in range(_RCPG):
                for v in range(VPC):
                    sl = pl.ds(v * LANES, LANES)
                    s = sbuf[c, sl]
                    d = dbuf[c, sl]
                    plsc.addupdate_scatter(
                        accl, [jnp.where(s != d, s, DEAD)], ones)
            return carry
        lax.fori_loop(0, _RGPT, group, 0)
        pltpu.sync_copy(accl, stage.at[sid])

    plsc.subcore_barrier()

    @pl.when(on0)
    def _drain():
        r0 = sid * RPT
        _reduce_stage(stage, abuf, tbuf, r0)
        pltpu.sync_copy(abuf, deg_hbm.at[pl.ds(r0, RPT)])


_B_NW = 28
_B_CPW = NCHUNK // _B_NW
_B_CPG = 8
_B_GPW = _B_CPW // _B_CPG
_B_NPW = NPAD // (NCORE * NTILE)

@functools.partial(
    pl.kernel,
    out_type=(jax.ShapeDtypeStruct((NQ * NCHUNK, CHUNK), _i32),
              jax.ShapeDtypeStruct((NCHUNK, CHUNK), _i32),
              jax.ShapeDtypeStruct((NPAD,), _f32)),
    mesh=_MESH,
    name="sc_prep",
    compiler_params=_SC_PARAMS,
    scratch_types=[
        pltpu.VMEM((_B_CPG, CHUNK), _i32),
        pltpu.VMEM((_B_CPG, CHUNK), _i32),
        pltpu.VMEM((_B_CPG, CHUNK), _i32),
        pltpu.VMEM((_B_NPW,), _f32),
        pltpu.VMEM((_B_NPW,), _f32),
        pltpu.VMEM((_B_NPW,), _f32),
    ],
)
def _prep_kernel(src_hbm, dst_hbm, dis_hbm, x_hbm,
                 srcef_hbm, dstef_hbm, u0_hbm,
                 sbuf, dbuf, hbuf, xbuf, disbuf, ubuf):
    cid = lax.axis_index("c")
    sid = lax.axis_index("s")
    wid = sid * NCORE + cid

    @pl.when(wid < _B_NW)
    def _edges():
        def group(g, carry):
            base = wid * _B_CPW + g * _B_CPG
            pltpu.sync_copy(src_hbm.at[pl.ds(base, _B_CPG)], sbuf)
            pltpu.sync_copy(dst_hbm.at[pl.ds(base, _B_CPG)], dbuf)
            for qq in range(NQ):
                for c in range(_B_CPG):
                    for v in range(VPC):
                        sl = pl.ds(v * LANES, LANES)
                        hbuf[c, sl] = sbuf[c, sl] * NQ + qq
                pltpu.sync_copy(
                    hbuf, srcef_hbm.at[pl.ds(qq * NCHUNK + base, _B_CPG)])
            for c in range(_B_CPG):
                for v in range(VPC):
                    sl = pl.ds(v * LANES, LANES)
                    s = sbuf[c, sl]
                    d = dbuf[c, sl]
                    dbuf[c, sl] = jnp.where(s != d, d, DEAD)
            pltpu.sync_copy(dbuf, dstef_hbm.at[pl.ds(base, _B_CPG)])
            return carry
        lax.fori_loop(0, _B_GPW, group, 0)

    nbase = wid * _B_NPW
    pltpu.sync_copy(x_hbm.at[pl.ds(nbase, _B_NPW)], xbuf)
    pltpu.sync_copy(dis_hbm.at[pl.ds(nbase, _B_NPW)], disbuf)
    def nrow(r, carry):
        sl = pl.ds(r * LANES, LANES)
        ubuf[sl] = xbuf[sl] * disbuf[sl]
        return carry
    lax.fori_loop(0, _B_NPW // LANES, nrow, 0)
    pltpu.sync_copy(ubuf, u0_hbm.at[pl.ds(nbase, _B_NPW)])


def _make_p1(first):
    @functools.partial(
        pl.kernel,
        out_type=(jax.ShapeDtypeStruct((NPAD,), _f32),
                  jax.ShapeDtypeStruct((NPAD,), _f32),
                  jax.ShapeDtypeStruct((NTILE, NPAD), _f32)),
        mesh=_MESH,
        name="sc_p1f" if first else "sc_p1r",
        compiler_params=_SC_PARAMS,
        scratch_types=[
            pltpu.VMEM((NPAD,), _f32),
            pltpu.VMEM((NPAD,), _f32),
            pltpu.VMEM((_RCPG, CHUNK), _i32),
            pltpu.VMEM((_RCPG, CHUNK), _i32),
            pltpu.VMEM((RPT,), _f32),
            pltpu.VMEM((RPT,), _f32),
            pltpu.VMEM((RPT,), _f32),
        ],
    )
    def p1(u_hbm, src_hbm, dstef_hbm, dis_hbm, t0_hbm, tx_hbm, uo_hbm, stage,
           utab, accl, sbuf, dbuf, abuf, tbuf, disb):
        cid = lax.axis_index("c")
        sid = lax.axis_index("s")
        on0 = cid == 0

        @pl.when(on0)
        def _main():
            _zero_vec(accl, NPAD)
            pltpu.sync_copy(u_hbm, utab)
            def group(g, carry):
                base = sid * CPT + g * _RCPG
                pltpu.sync_copy(src_hbm.at[pl.ds(base, _RCPG)], sbuf)
                pltpu.sync_copy(dstef_hbm.at[pl.ds(base, _RCPG)], dbuf)
                for c in range(_RCPG):
                    for v in range(VPC):
                        sl = pl.ds(v * LANES, LANES)
                        val = plsc.load_gather(utab, [sbuf[c, sl]])
                        plsc.addupdate_scatter(accl, [dbuf[c, sl]], val)
                return carry
            lax.fori_loop(0, _RGPT, group, 0)
            pltpu.sync_copy(accl, stage.at[sid])

        plsc.subcore_barrier()

        @pl.when(on0)
        def _drain():
            r0 = sid * RPT
            _reduce_stage(stage, abuf, tbuf, r0)
            pltpu.sync_copy(dis_hbm.at[pl.ds(r0, RPT)], disb)
            if not first:
                pltpu.sync_copy(t0_hbm.at[pl.ds(r0, RPT)], tbuf)
            def drow(r, carry):
                sl = pl.ds(r * LANES, LANES)
                a = abuf[sl]
                d = disb[sl]
                if first:
                    t = -(a * d)
                else:
                    t = (a * d) * (-2.0) - tbuf[sl]
                abuf[sl] = t
                disb[sl] = t * d
                return carry
            lax.fori_loop(0, RPT // LANES, drow, 0)
            pltpu.sync_copy(abuf, tx_hbm.at[pl.ds(r0, RPT)])
            pltpu.sync_copy(disb, uo_hbm.at[pl.ds(r0, RPT)])

    return p1


_P1_FIRST = _make_p1(True)
_P1_REST = _make_p1(False)


def _make_p32(first):
    @functools.partial(
        pl.kernel,
        out_type=(jax.ShapeDtypeStruct((NPAD, 64), _f32),
                  jax.ShapeDtypeStruct((NPAD, 64), _f32)),
        mesh=_MESH,
        name="sc_p32f" if first else "sc_p32r",
        compiler_params=_SC_PARAMS,
        scratch_types=[
            pltpu.VMEM((4, CPG, CHUNK), _i32),
            pltpu.VMEM((4, CPG, CHUNK), _i32),
            pltpu.VMEM((4, CPG, CHUNK, QCOL), _f32),
            pltpu.VMEM((DR32, QCOL), _f32),
            pltpu.VMEM((DR32, QCOL), _f32),
            pltpu.VMEM((DR32, QCOL), _f32),
            pltpu.VMEM((DR32, QCOL), _f32),
            pltpu.VMEM((DR32, QCOL), _f32),
            pltpu.VMEM_SHARED((NPAD, QCOL), _f32),
            pltpu.SemaphoreType.DMA,
            pltpu.SemaphoreType.DMA,
            pltpu.SemaphoreType.DMA,
            pltpu.SemaphoreType.DMA,
            pltpu.SemaphoreType.DMA,
            pltpu.SemaphoreType.DMA,
        ],
    )
    def p32(u_hbm, srcef_hbm, dstef_hbm, disb_hbm, t0_hbm, tx_hbm, uo_hbm,
            sbuf, dbuf, rbuf, abuf, t0b, dbb, txb, uob, acc,
            sem_i0, sem_i1, sem_g0, sem_g1, sem_s0, sem_s1):
        cid = lax.axis_index("c")
        sid = lax.axis_index("s")
        sem_i = (sem_i0, sem_i1)
        sem_g = (sem_g0, sem_g1)
        sem_s = (sem_s0, sem_s1)

        def idx_load(qq, g, sl, par, sync=False):
            sbase = qq * NCHUNK + sid * CPT + g * CPG
            dbase = sid * CPT + g * CPG
            if sync:
                pltpu.sync_copy(srcef_hbm.at[pl.ds(sbase, CPG)], sbuf.at[sl])
                pltpu.sync_copy(dstef_hbm.at[pl.ds(dbase, CPG)], dbuf.at[sl])
            else:
                pltpu.async_copy(srcef_hbm.at[pl.ds(sbase, CPG)],
                                 sbuf.at[sl], sem_i[par])
                pltpu.async_copy(dstef_hbm.at[pl.ds(dbase, CPG)],
                                 dbuf.at[sl], sem_i[par])

        def wait_idx(par):
            for _ in range(2):
                pltpu.make_async_copy(
                    srcef_hbm.at[pl.ds(0, CPG)], sbuf.at[0],
                    sem_i[par]).wait()

        def fire_gathers(sl, par):
            for c in range(CPG):
                pltpu.async_copy(u_hbm.at[sbuf.at[sl, c]],
                                 rbuf.at[sl, c], sem_g[par])

        def wait_gathers(par):
            for _ in range(CPG):
                pltpu.make_async_copy(
                    u_hbm.at[sbuf.at[0, 0]], rbuf.at[0, 0],
                    sem_g[par]).wait()

        def fire_scatters(sl, par):
            for c in range(CPG):
                pltpu.async_copy(rbuf.at[sl, c], acc.at[dbuf.at[sl, c]],
                                 sem_s[par], add=True)

        def wait_scatters(par):
            for _ in range(CPG):
                pltpu.make_async_copy(
                    rbuf.at[0, 0], acc.at[dbuf.at[0, 0]], sem_s[par]).wait()

        for q in range(NCORE):
            qq = cid * NCORE + q

            _zero_fill(abuf, DR32, QCOL)
            for it in range(RPT // DR32):
                pltpu.sync_copy(
                    abuf, acc.at[pl.ds(sid * RPT + it * DR32, DR32)])

            plsc.subcore_barrier()

            def ring_iter(g, m4, do_ws, do_idx, do_fg):
                m2 = m4 % 2
                if do_ws:
                    wait_scatters(m2)
                if do_idx:
                    idx_load(qq, g + 2, (m4 + 2) % 4, m2)
                if do_fg:
                    wait_idx((m2 + 1) % 2)
                    fire_gathers((m4 + 1) % 4, (m2 + 1) % 2)
                wait_gathers(m2)
                fire_scatters(m4, m2)

            idx_load(qq, 0, 0, 0, sync=True)
            idx_load(qq, 1, 1, 1)
            fire_gathers(0, 0)
            ring_iter(0, 0, False, True, True)
            ring_iter(1, 1, False, True, True)
            def quad(j, carry):
                g = 4 * j + 2
                for u in range(4):
                    ring_iter(g + u, (2 + u) % 4, True, True, True)
                return carry
            lax.fori_loop(0, (GPT - 6) // 4, quad, 0)
            ring_iter(GPT - 4, (GPT - 4) % 4, True, True, True)
            ring_iter(GPT - 3, (GPT - 3) % 4, True, True, True)
            ring_iter(GPT - 2, (GPT - 2) % 4, True, False, True)
            ring_iter(GPT - 1, (GPT - 1) % 4, True, False, False)
            wait_scatters((GPT - 2) % 2)
            wait_scatters((GPT - 1) % 2)

            plsc.subcore_barrier()

            def drain(it, carry):
                rloc = sid * RPT + it * DR32
                csl = pl.ds(qq * QCOL, QCOL)
                pltpu.sync_copy(acc.at[pl.ds(rloc, DR32)], abuf)
                pltpu.sync_copy(disb_hbm.at[pl.ds(rloc, DR32)], dbb)
                if not first:
                    pltpu.sync_copy(t0_hbm.at[pl.ds(rloc, DR32), csl], t0b)
                def drow(r, carry2):
                    sl = pl.ds(0, LANES)
                    a = abuf[r, sl]
                    d = dbb[r, sl]
                    if first:
                        t = -(a * d)
                    else:
                        t = (a * d) * (-2.0) - t0b[r, sl]
                    txb[r, sl] = t
                    uob[r, sl] = t * d
                    return carry2
                lax.fori_loop(0, DR32, drow, 0)
                pltpu.sync_copy(txb, tx_hbm.at[pl.ds(rloc, DR32), csl])
                pltpu.sync_copy(uob, uo_hbm.at[pl.ds(rloc, DR32), csl])
                return carry
            lax.fori_loop(0, RPT // DR32, drain, 0)

            if q == 0:
                plsc.subcore_barrier()

    return p32


_P32_FIRST = _make_p32(True)
_P32_REST = _make_p32(False)


_BLK = 512
_NB = NPAD // _BLK


def _make_tc_mid(nk):
    def body(*refs):
        tx_refs = refs[:nk]
        w_ref, b_ref, dis_ref, h_ref, u_ref = refs[nk:]
        acc = b_ref[...].astype(_f32) * jnp.ones((_BLK, 1), _f32)
        for k in range(nk):
            acc = acc + jnp.dot(tx_refs[k][...],
                                w_ref[pl.ds(k * 64, 64), :],
                                preferred_element_type=_f32,
                                precision=lax.Precision.HIGHEST)
        h = jnp.maximum(acc, 0.0)
        h_ref[...] = h
        u_ref[...] = h * dis_ref[...]
    return body


def _tc_mid(txs, w_r, b, dis2d):
    nk = len(txs)
    h, u = pl.pallas_call(
        _make_tc_mid(nk),
        grid=(_NB,),
        in_specs=(
            [pl.BlockSpec((_BLK, 64), lambda i: (i, 0))] * nk
            + [pl.BlockSpec((nk * 64, 64), lambda i: (0, 0)),
               pl.BlockSpec((1, 64), lambda i: (0, 0)),
               pl.BlockSpec((_BLK, 1), lambda i: (i, 0))]
        ),
        out_specs=[pl.BlockSpec((_BLK, 64), lambda i: (i, 0))] * 2,
        out_shape=[jax.ShapeDtypeStruct((NPAD, 64), _f32)] * 2,
    )(*txs, w_r, b.reshape(1, 64), dis2d)
    return h, u


def _tc1_body(tx_ref, w_ref, b_ref, dis_ref, h_ref, u_ref):
    h = jnp.dot(tx_ref[...], w_ref[...],
                preferred_element_type=_f32,
                precision=lax.Precision.HIGHEST)
    h = jnp.maximum(h + b_ref[...], 0.0)
    h_ref[...] = h
    u_ref[...] = h * dis_ref[...]


def _tc1(txs1, w_r, b, dis2d):
    h, u = pl.pallas_call(
        _tc1_body,
        grid=(_NB,),
        in_specs=[
            pl.BlockSpec((_BLK, KORD), lambda i: (i, 0)),
            pl.BlockSpec((KORD, 64), lambda i: (0, 0)),
            pl.BlockSpec((1, 64), lambda i: (0, 0)),
            pl.BlockSpec((_BLK, 1), lambda i: (i, 0)),
        ],
        out_specs=[pl.BlockSpec((_BLK, 64), lambda i: (i, 0))] * 2,
        out_shape=[jax.ShapeDtypeStruct((NPAD, 64), _f32)] * 2,
    )(txs1, w_r, b.reshape(1, 64), dis2d)
    return h, u


def _make_tc_final(nk):
    def body(*refs):
        tx_refs = refs[:nk]
        w_ref, b_ref, fcw_ref, fcb_ref, o_ref = refs[nk:]
        acc = b_ref[...].astype(_f32) * jnp.ones((_BLK, 1), _f32)
        for k in range(nk):
            acc = acc + jnp.dot(tx_refs[k][...],
                                w_ref[pl.ds(k * 64, 64), :],
                                preferred_element_type=_f32,
                                precision=lax.Precision.HIGHEST)
        h = jnp.maximum(acc, 0.0)
        o_ref[...] = jnp.dot(h, fcw_ref[...],
                             preferred_element_type=_f32,
                             precision=lax.Precision.HIGHEST) + fcb_ref[...]
    return body


def _tc_final(txs, w_r, b, fcw, fcb):
    nk = len(txs)
    return pl.pallas_call(
        _make_tc_final(nk),
        grid=(_NB,),
        in_specs=(
            [pl.BlockSpec((_BLK, 64), lambda i: (i, 0))] * nk
            + [pl.BlockSpec((nk * 64, 64), lambda i: (0, 0)),
               pl.BlockSpec((1, 64), lambda i: (0, 0)),
               pl.BlockSpec((64, 1), lambda i: (0, 0)),
               pl.BlockSpec((1, 1), lambda i: (0, 0))]
        ),
        out_specs=pl.BlockSpec((_BLK, 1), lambda i: (i, 0)),
        out_shape=jax.ShapeDtypeStruct((NPAD, 1), _f32),
    )(*txs, w_r, b.reshape(1, 64), fcw, fcb.reshape(1, 1))


def _layer32(u_tbl, tx0_tbl, srcef, dstef, disb, dis2d, w, b,
             final_args=None):
    qview = lambda t: t.reshape(NQ * NPAD, QCOL)
    t1, v1 = _P32_FIRST(qview(u_tbl), srcef, dstef, disb, tx0_tbl)
    t2, v2 = _P32_REST(qview(v1), srcef, dstef, disb, tx0_tbl)
    t3, v3 = _P32_REST(qview(v2), srcef, dstef, disb, t1)
    t4, _ = _P32_REST(qview(v3), srcef, dstef, disb, t2)
    txs = [tx0_tbl, t1, t2, t3, t4]
    w_r = w.reshape(KORD * 64, 64)
    if final_args is None:
        return _tc_mid(txs, w_r, b, dis2d)
    fcw, fcb = final_args
    return _tc_final(txs, w_r, b, fcw, fcb)


def kernel(x, edge_index, W1, b1, W2, b2, W3, b3, fcw, fcb):
    src = jnp.pad(edge_index[0], (0, EPAD - NEDGE)).reshape(NCHUNK, CHUNK)
    dst = jnp.pad(edge_index[1], (0, EPAD - NEDGE)).reshape(NCHUNK, CHUNK)
    x_pad = jnp.pad(x[:, 0], (0, NPAD - NNODE))

    deg, _ = _deg_kernel(src, dst)
    dis = jnp.where(deg > 0, lax.rsqrt(jnp.maximum(deg, 1e-30)), 0.0)
    dis2d = dis[:, None]
    disb = jnp.broadcast_to(dis2d, (NPAD, QCOL))

    srcef, dstef, u0 = _prep_kernel(src, dst, dis, x_pad)

    t1, v1, _ = _P1_FIRST(u0, src, dstef, dis, u0)
    t2, v2, _ = _P1_REST(v1, src, dstef, dis, x_pad)
    t3, v3, _ = _P1_REST(v2, src, dstef, dis, t1)
    t4, _, _ = _P1_REST(v3, src, dstef, dis, t2)
    txs1 = jnp.stack([x_pad, t1, t2, t3, t4], axis=1)

    h1, u1 = _tc1(txs1, W1.reshape(KORD, 64), b1, dis2d)

    h2, u2 = _layer32(u1, h1, srcef, dstef, disb, dis2d, W2, b2)

    y = _layer32(u2, h2, srcef, dstef, disb, dis2d, W3, b3,
                 final_args=(fcw, fcb))
    return y[:NNODE]
